# baseline (speedup 1.0000x reference)
import jax
import jax.numpy as jnp
from jax.experimental import pallas as pl
from jax.experimental.pallas import tpu as pltpu

MAP_UNITS = 2048
IN_CH = 256
BATCH = 8
DIST_THRESHOLD = 200.0
ROW_BLK = 256
N_BLOCKS = MAP_UNITS // ROW_BLK


def _fused_kernel(d_ref, x_ref, wc_ref, b_ref, y_ref,
                  edge_sc, x_sc, deg_sc, u_sc, z_sc, w_sc, gcn_sc):
    s = pl.program_id(0)

    @pl.when(s < N_BLOCKS)
    def _ingest():
        j = s
        d = d_ref[...]
        edge = jnp.where(d < DIST_THRESHOLD, d, 0.0)
        ones = jnp.ones((1, ROW_BLK), dtype=jnp.float32)
        deg_part = jax.lax.dot_general(
            ones, edge, (((1,), (0,)), ((), ())),
            preferred_element_type=jnp.float32)

        @pl.when(j == 0)
        def _():
            deg_sc[...] = deg_part

        @pl.when(j > 0)
        def _():
            deg_sc[...] += deg_part

        edge_sc[pl.ds(j * ROW_BLK, ROW_BLK), :] = edge.astype(jnp.bfloat16)

        x = x_ref[...]
        x_sc[:, pl.ds(j * ROW_BLK, ROW_BLK), :] = x
        wc = wc_ref[...]
        zu = jax.lax.dot_general(
            x, wc, (((2,), (0,)), ((), ())),
            preferred_element_type=jnp.float32)
        u_sc[:, pl.ds(j * ROW_BLK, ROW_BLK)] = zu[:, :, 0]
        z_sc[:, pl.ds(j * ROW_BLK, ROW_BLK)] = zu[:, :, 1]

    e = s - N_BLOCKS
    nj = e // BATCH
    bb = e % BATCH

    @pl.when(s == N_BLOCKS)
    def _prep():
        deg = deg_sc[...]
        dinv = jnp.where(deg > 0.0, jax.lax.rsqrt(deg), 0.0)
        w_sc[...] = (z_sc[...] * dinv).astype(jnp.bfloat16)

    @pl.when((s >= N_BLOCKS) & (bb == 0))
    def _matvec_chunk():
        w = w_sc[...]
        edge = edge_sc[pl.ds(nj * ROW_BLK, ROW_BLK), :]
        t = jax.lax.dot_general(
            w, edge.astype(jnp.bfloat16), (((1,), (1,)), ((), ())),
            preferred_element_type=jnp.float32)
        deg_n = deg_sc[0, pl.ds(nj * ROW_BLK, ROW_BLK)]
        dinv_n = jnp.where(deg_n > 0.0, jax.lax.rsqrt(deg_n), 0.0)
        u = u_sc[:, pl.ds(nj * ROW_BLK, ROW_BLK)]
        out = u - dinv_n[None, :] * t + b_ref[0, 0]
        gcn_sc[:, pl.ds(nj * ROW_BLK, ROW_BLK)] = jax.nn.sigmoid(out)

    @pl.when(s >= N_BLOCKS)
    def _emit():
        y_ref[0, :, 0:IN_CH] = x_sc[bb, pl.ds(nj * ROW_BLK, ROW_BLK), :]
        y_ref[0, :, IN_CH:IN_CH + 1] = (
            gcn_sc[bb, pl.ds(nj * ROW_BLK, ROW_BLK)][:, None])


@jax.jit
def kernel(x, dist_mat, W, b):
    wc = jnp.concatenate([W[0], W[1]], axis=1)
    b2 = jnp.reshape(b, (1, 1)).astype(jnp.float32)

    def _ingest_idx(s):
        return jnp.minimum(s, N_BLOCKS - 1)

    def _out_idx(s):
        e = jnp.maximum(s - N_BLOCKS, 0)
        return (e % BATCH, e // BATCH, 0)

    y = pl.pallas_call(
        _fused_kernel,
        grid=(N_BLOCKS + N_BLOCKS * BATCH,),
        in_specs=[
            pl.BlockSpec((ROW_BLK, MAP_UNITS), lambda s: (_ingest_idx(s), 0)),
            pl.BlockSpec((BATCH, ROW_BLK, IN_CH),
                         lambda s: (0, _ingest_idx(s), 0)),
            pl.BlockSpec((IN_CH, 2), lambda s: (0, 0)),
            pl.BlockSpec((1, 1), lambda s: (0, 0)),
        ],
        out_specs=pl.BlockSpec((1, ROW_BLK, IN_CH + 1), _out_idx),
        out_shape=jax.ShapeDtypeStruct(
            (BATCH, MAP_UNITS, IN_CH + 1), jnp.float32),
        scratch_shapes=[
            pltpu.VMEM((MAP_UNITS, MAP_UNITS), jnp.bfloat16),
            pltpu.VMEM((BATCH, MAP_UNITS, IN_CH), jnp.float32),
            pltpu.VMEM((1, MAP_UNITS), jnp.float32),
            pltpu.VMEM((BATCH, MAP_UNITS), jnp.float32),
            pltpu.VMEM((BATCH, MAP_UNITS), jnp.float32),
            pltpu.VMEM((BATCH, MAP_UNITS), jnp.bfloat16),
            pltpu.VMEM((BATCH, MAP_UNITS), jnp.float32),
        ],
    )(dist_mat, x, wc, b2)

    return y


# speedup vs baseline: 1.5176x; 1.5176x over previous
import jax
import jax.numpy as jnp
from jax.experimental import pallas as pl
from jax.experimental.pallas import tpu as pltpu

MAP_UNITS = 2048
IN_CH = 256
BATCH = 8
DIST_THRESHOLD = 200.0
ROW_BLK = 256
N_BLOCKS = MAP_UNITS // ROW_BLK


def _fused_kernel(d_ref, x_ref, wt_ref, b_ref, y_ref,
                  x_sc, deg_sc, u_sc, t_sc, gcn_sc):
    s = pl.program_id(0)

    @pl.when(s < N_BLOCKS)
    def _ingest():
        j = s
        d = d_ref[...]
        edge = jnp.where(d < DIST_THRESHOLD, d, 0.0)
        deg_row = jnp.sum(edge, axis=1, keepdims=True)
        dinv_row = jnp.where(deg_row > 0.0, jax.lax.rsqrt(deg_row), 0.0)
        edge_w = (edge * dinv_row).astype(jnp.bfloat16)
        deg_part = jnp.sum(edge, axis=0, keepdims=True)

        x = x_ref[...]
        x_sc[:, pl.ds(j * ROW_BLK, ROW_BLK), :] = x
        wt = wt_ref[...]
        zu = jax.lax.dot_general(
            wt, x, (((1,), (2,)), ((), ())),
            preferred_element_type=jnp.float32)
        u_sc[:, pl.ds(j * ROW_BLK, ROW_BLK)] = zu[0]
        zw = zu[1].astype(jnp.bfloat16)

        t_part = jax.lax.dot_general(
            zw, edge_w, (((1,), (0,)), ((), ())),
            preferred_element_type=jnp.float32)

        @pl.when(j == 0)
        def _():
            deg_sc[...] = deg_part
            t_sc[...] = t_part

        @pl.when(j > 0)
        def _():
            deg_sc[...] += deg_part
            t_sc[...] += t_part

    @pl.when(s == N_BLOCKS)
    def _finalize():
        deg = deg_sc[...]
        dinv = jnp.where(deg > 0.0, jax.lax.rsqrt(deg), 0.0)
        out = u_sc[...] - dinv * t_sc[...] + b_ref[0, 0]
        gcn_sc[...] = jax.nn.sigmoid(out)

    @pl.when(s >= N_BLOCKS)
    def _emit():
        bb = s - N_BLOCKS
        y_ref[0, :, 0:IN_CH] = x_sc[bb]
        y_ref[0, :, IN_CH:IN_CH + 1] = gcn_sc[bb][:, None]


@jax.jit
def kernel(x, dist_mat, W, b):
    wt = W[:, :, 0]
    b2 = jnp.reshape(b, (1, 1)).astype(jnp.float32)

    def _ingest_idx(s):
        return jnp.minimum(s, N_BLOCKS - 1)

    y = pl.pallas_call(
        _fused_kernel,
        grid=(2 * N_BLOCKS,),
        in_specs=[
            pl.BlockSpec((ROW_BLK, MAP_UNITS), lambda s: (_ingest_idx(s), 0)),
            pl.BlockSpec((BATCH, ROW_BLK, IN_CH),
                         lambda s: (0, _ingest_idx(s), 0)),
            pl.BlockSpec((2, IN_CH), lambda s: (0, 0)),
            pl.BlockSpec((1, 1), lambda s: (0, 0)),
        ],
        out_specs=pl.BlockSpec(
            (1, MAP_UNITS, IN_CH + 1),
            lambda s: (jnp.maximum(s - N_BLOCKS, 0), 0, 0)),
        out_shape=jax.ShapeDtypeStruct(
            (BATCH, MAP_UNITS, IN_CH + 1), jnp.float32),
        scratch_shapes=[
            pltpu.VMEM((BATCH, MAP_UNITS, IN_CH), jnp.float32),
            pltpu.VMEM((1, MAP_UNITS), jnp.float32),
            pltpu.VMEM((BATCH, MAP_UNITS), jnp.float32),
            pltpu.VMEM((BATCH, MAP_UNITS), jnp.float32),
            pltpu.VMEM((BATCH, MAP_UNITS), jnp.float32),
        ],
    )(dist_mat, x, wt, b2)

    return y


# speedup vs baseline: 1.5502x; 1.0215x over previous
import jax
import jax.numpy as jnp
from jax.experimental import pallas as pl
from jax.experimental.pallas import tpu as pltpu

MAP_UNITS = 2048
IN_CH = 256
BATCH = 8
DIST_THRESHOLD = 200.0
ROW_BLK = 512
N_BLOCKS = MAP_UNITS // ROW_BLK


def _fused_kernel(d_ref, x_ref, wt_ref, b_ref, y_ref,
                  x_sc, deg_sc, u_sc, t_sc, gcn_sc):
    s = pl.program_id(0)

    @pl.when(s < N_BLOCKS)
    def _ingest():
        j = s
        d = d_ref[...]
        edge = jnp.where(d < DIST_THRESHOLD, d, 0.0)
        deg_row = jnp.sum(edge, axis=1, keepdims=True)
        dinv_row = jnp.where(deg_row > 0.0, jax.lax.rsqrt(deg_row), 0.0)
        edge_w = (edge * dinv_row).astype(jnp.bfloat16)
        deg_part = jnp.sum(edge, axis=0, keepdims=True)

        x = x_ref[...]
        x_sc[:, pl.ds(j * ROW_BLK, ROW_BLK), :] = x
        wt = wt_ref[...]
        zu = jax.lax.dot_general(
            wt, x, (((1,), (2,)), ((), ())),
            preferred_element_type=jnp.float32)
        u_sc[:, pl.ds(j * ROW_BLK, ROW_BLK)] = zu[0]
        zw = zu[1].astype(jnp.bfloat16)

        t_part = jax.lax.dot_general(
            zw, edge_w, (((1,), (0,)), ((), ())),
            preferred_element_type=jnp.float32)

        @pl.when(j == 0)
        def _():
            deg_sc[...] = deg_part
            t_sc[...] = t_part

        @pl.when(j > 0)
        def _():
            deg_sc[...] += deg_part
            t_sc[...] += t_part

    @pl.when(s == N_BLOCKS)
    def _finalize():
        deg = deg_sc[...]
        dinv = jnp.where(deg > 0.0, jax.lax.rsqrt(deg), 0.0)
        out = u_sc[...] - dinv * t_sc[...] + b_ref[0, 0]
        gcn_sc[...] = jax.nn.sigmoid(out)

    @pl.when(s >= N_BLOCKS)
    def _emit():
        bb = s - N_BLOCKS
        y_ref[0, :, 0:IN_CH] = x_sc[bb]
        y_ref[0, :, IN_CH:IN_CH + 1] = gcn_sc[bb][:, None]


@jax.jit
def kernel(x, dist_mat, W, b):
    wt = W[:, :, 0]
    b2 = jnp.reshape(b, (1, 1)).astype(jnp.float32)

    def _ingest_idx(s):
        return jnp.minimum(s, N_BLOCKS - 1)

    y = pl.pallas_call(
        _fused_kernel,
        grid=(N_BLOCKS + BATCH,),
        in_specs=[
            pl.BlockSpec((ROW_BLK, MAP_UNITS), lambda s: (_ingest_idx(s), 0)),
            pl.BlockSpec((BATCH, ROW_BLK, IN_CH),
                         lambda s: (0, _ingest_idx(s), 0)),
            pl.BlockSpec((2, IN_CH), lambda s: (0, 0)),
            pl.BlockSpec((1, 1), lambda s: (0, 0)),
        ],
        out_specs=pl.BlockSpec(
            (1, MAP_UNITS, IN_CH + 1),
            lambda s: (jnp.maximum(s - N_BLOCKS, 0), 0, 0)),
        out_shape=jax.ShapeDtypeStruct(
            (BATCH, MAP_UNITS, IN_CH + 1), jnp.float32),
        scratch_shapes=[
            pltpu.VMEM((BATCH, MAP_UNITS, IN_CH), jnp.float32),
            pltpu.VMEM((1, MAP_UNITS), jnp.float32),
            pltpu.VMEM((BATCH, MAP_UNITS), jnp.float32),
            pltpu.VMEM((BATCH, MAP_UNITS), jnp.float32),
            pltpu.VMEM((BATCH, MAP_UNITS), jnp.float32),
        ],
    )(dist_mat, x, wt, b2)

    return y
